# baseline (speedup 1.0000x reference)
import functools

import jax
import jax.numpy as jnp
from jax import lax
from jax.experimental import pallas as pl
from jax.experimental.pallas import tpu as pltpu
from jax.experimental.pallas import tpu_sc as plsc

_N = 10000
_E = 320000
_NC = 2
_NS = 16
_NW = _NC * _NS
_G = 40
_EPT = _E // _NW
_J = _EPT // _G
_NB = 10
_JO = _J // _NB
_NPAD = 10112
_RPT = _NPAD // _NS


def _sc_mesh():
    return plsc.VectorSubcoreMesh(core_axis_name="c", subcore_axis_name="s")


_NB0 = 5
_G0 = _G
_J0 = _J
_JO0 = _J0 // _NB0


def _sc_pass0(edge_h, dstg, zeros_h, zeros_d, ones_g):
    @functools.partial(
        pl.kernel,
        out_type=(
            jax.ShapeDtypeStruct((_NC, _NPAD, 128), jnp.float32),
            jax.ShapeDtypeStruct((_NC, _NPAD, 16), jnp.float32),
        ),
        mesh=_sc_mesh(),
        scratch_types=(
            pltpu.VMEM((_J0, _G0), jnp.int32),
            pltpu.VMEM((_NB0, _G0, 128), jnp.float32),
            pltpu.VMEM((_G0, 16), jnp.float32),
            pltpu.VMEM_SHARED((_NPAD, 128), jnp.float32),
            pltpu.VMEM_SHARED((_NPAD, 16), jnp.float32),
            pltpu.SemaphoreType.DMA((_NB0,)),
            pltpu.SemaphoreType.DMA((_NB0,)),
            pltpu.SemaphoreType.DMA((_NB0,)),
        ),
        compiler_params=pltpu.CompilerParams(use_tc_tiling_on_sc=False),
    )
    def k(eh, dg, zh, zd, og, oh, od, idxv, ebuf, ones_v, acc_h, acc_d,
          gsem, ssem, dsem):
        cid = lax.axis_index("c")
        sid = lax.axis_index("s")
        wid = cid * _NS + sid
        pltpu.sync_copy(zh, acc_h.at[pl.ds(sid * _RPT, _RPT), :])
        pltpu.sync_copy(zd, acc_d.at[pl.ds(sid * _RPT, _RPT), :])
        pltpu.sync_copy(og, ones_v)
        pltpu.sync_copy(dg.at[wid], idxv)
        plsc.subcore_barrier()

        def edge_dma(j, b):
            base = wid * _EPT + j * _G0
            return pltpu.make_async_copy(
                eh.at[pl.ds(base, _G0), :], ebuf.at[b], gsem.at[b])

        def scat(j, b):
            return (pltpu.make_async_copy(ebuf.at[b], acc_h.at[idxv.at[j]],
                                          ssem.at[b]),
                    pltpu.make_async_copy(ones_v, acc_d.at[idxv.at[j]],
                                          dsem.at[b]))

        for b in range(_NB0):
            edge_dma(b, b).start()

        def body(jo, carry):
            j0 = jo * _NB0
            for b in range(_NB0):
                j = j0 + b
                edge_dma(j, b).wait()
                s1, s2 = scat(j, b)
                s1.start(add=True)
                s2.start(add=True)
            for b in range(_NB0):
                j = j0 + b
                s1, s2 = scat(j, b)
                s1.wait()
                s2.wait()

                @pl.when(jo < _JO0 - 1)
                def _():
                    edge_dma(j + _NB0, b).start()

            return carry

        lax.fori_loop(0, _JO0, body, 0)
        plsc.subcore_barrier()
        pltpu.sync_copy(acc_h.at[pl.ds(sid * _RPT, _RPT), :],
                        oh.at[cid, pl.ds(sid * _RPT, _RPT), :])
        pltpu.sync_copy(acc_d.at[pl.ds(sid * _RPT, _RPT), :],
                        od.at[cid, pl.ds(sid * _RPT, _RPT), :])

    return k(edge_h, dstg, zeros_h, zeros_d, ones_g)


def _sc_gather_scatter(t_nodes, srcg, dstg, zeros_f, f):
    @functools.partial(
        pl.kernel,
        out_type=jax.ShapeDtypeStruct((_NC, _NPAD, f), jnp.float32),
        mesh=_sc_mesh(),
        scratch_types=(
            pltpu.VMEM((_J, _G), jnp.int32),
            pltpu.VMEM((_J, _G), jnp.int32),
            pltpu.VMEM((_NB, _G, f), jnp.float32),
            pltpu.VMEM_SHARED((_NPAD, f), jnp.float32),
            pltpu.SemaphoreType.DMA((_NB,)),
            pltpu.SemaphoreType.DMA((_NB,)),
        ),
        compiler_params=pltpu.CompilerParams(use_tc_tiling_on_sc=False),
    )
    def k(tn, sg, dg, zf, om, sidxv, didxv, rbuf, acc, gsem, ssem):
        cid = lax.axis_index("c")
        sid = lax.axis_index("s")
        wid = cid * _NS + sid
        pltpu.sync_copy(zf, acc.at[pl.ds(sid * _RPT, _RPT), :])
        pltpu.sync_copy(sg.at[wid], sidxv)
        pltpu.sync_copy(dg.at[wid], didxv)
        plsc.subcore_barrier()

        def gather(j, b):
            return pltpu.make_async_copy(
                tn.at[sidxv.at[j]], rbuf.at[b], gsem.at[b])

        for b in range(_NB):
            gather(b, b).start()

        def body(jo, carry):
            j0 = jo * _NB
            for b in range(_NB):
                j = j0 + b
                gather(j, b).wait()
                pltpu.async_copy(rbuf.at[b], acc.at[didxv.at[j]],
                                 ssem.at[b], add=True)
            for b in range(_NB):
                j = j0 + b
                pltpu.make_async_copy(rbuf.at[b], acc.at[didxv.at[j]],
                                      ssem.at[b]).wait()

                @pl.when(jo < _JO - 1)
                def _():
                    gather(j + _NB, b).start()

            return carry

        lax.fori_loop(0, _JO, body, 0)
        plsc.subcore_barrier()
        pltpu.sync_copy(acc.at[pl.ds(sid * _RPT, _RPT), :],
                        om.at[cid, pl.ds(sid * _RPT, _RPT), :])

    return k(t_nodes, srcg, dstg, zeros_f)


def _tc_layer1(acc_h, acc_d, w1, b1, w2):
    def body(ah, ad, w1r, b1r, w2r, t1o, invo):
        s = ah[0, : _N, :] + ah[1, : _N, :]
        d = ad[0, : _N, 0:1] + ad[1, : _N, 0:1]
        inv = 1.0 / jnp.maximum(d, 1.0)
        h1 = jnp.maximum(
            jnp.dot(s * inv, w1r[...], preferred_element_type=jnp.float32)
            + b1r[...], 0.0)
        t1o[...] = jnp.dot(h1, w2r[...], preferred_element_type=jnp.float32)
        invo[...] = inv

    return pl.pallas_call(
        body,
        out_shape=(
            jax.ShapeDtypeStruct((_N, 32), jnp.float32),
            jax.ShapeDtypeStruct((_N, 1), jnp.float32),
        ),
    )(acc_h, acc_d, w1, b1, w2)


def _tc_mid(m, invd, b, wnext):
    f_out = wnext.shape[1]

    def body(mr, ir, br, wr, to):
        s = mr[0, : _N, :] + mr[1, : _N, :]
        h = jnp.maximum(s * ir[...] + br[...], 0.0)
        to[...] = jnp.dot(h, wr[...], preferred_element_type=jnp.float32)

    return pl.pallas_call(
        body,
        out_shape=jax.ShapeDtypeStruct((_N, f_out), jnp.float32),
    )(m, invd, b, wnext)


def _tc_layer3(m, invd, b3, wg1, bg1, wg2, bg2):
    def body(mr, ir, b3r, wg1r, bg1r, wg2r, bg2r, h3o, glo):
        s = mr[0, : _N, :] + mr[1, : _N, :]
        h3 = jnp.maximum(s * ir[...] + b3r[...], 0.0)
        h3o[...] = h3
        gm = jnp.mean(h3, axis=0, keepdims=True)
        z1 = jnp.maximum(
            jnp.dot(gm, wg1r[...], preferred_element_type=jnp.float32)
            + bg1r[...], 0.0)
        z2 = jnp.dot(z1, wg2r[...], preferred_element_type=jnp.float32) + bg2r[...]
        glo[...] = 1.0 / (1.0 + jnp.exp(-z2))

    return pl.pallas_call(
        body,
        out_shape=(
            jax.ShapeDtypeStruct((_N, 16), jnp.float32),
            jax.ShapeDtypeStruct((1, 1), jnp.float32),
        ),
    )(m, invd, b3, wg1, bg1, wg2, bg2)


def _tc_node(m, invd, wn, bn):
    def body(mr, ir, wr, br, no):
        s = mr[0, : _N, :] + mr[1, : _N, :]
        no[...] = (
            jnp.dot(s * ir[...], wr[...], preferred_element_type=jnp.float32)
            + br[...])

    return pl.pallas_call(
        body,
        out_shape=jax.ShapeDtypeStruct((_N, 2), jnp.float32),
    )(m, invd, wn, bn)


@jax.jit
def kernel(edge_h, edge_index, W1, b1, W2, b2, W3, b3, Wn, bn,
           Wg1, bg1, Wg2, bg2):
    srcg = edge_index[0].reshape(_NW, _J, _G)
    dstg = edge_index[1].reshape(_NW, _J, _G)
    dstg0 = dstg
    zeros_h = jnp.zeros((_RPT, 128), jnp.float32)
    zeros_d = jnp.zeros((_RPT, 16), jnp.float32)
    zeros_32 = jnp.zeros((_RPT, 32), jnp.float32)
    zeros_16 = jnp.zeros((_RPT, 16), jnp.float32)
    ones_g = jnp.ones((_G, 16), jnp.float32)

    acc_h, acc_d = _sc_pass0(edge_h, dstg0, zeros_h, zeros_d, ones_g)
    t1, invd = _tc_layer1(acc_h, acc_d, W1, b1.reshape(1, -1), W2)
    m1 = _sc_gather_scatter(t1, srcg, dstg, zeros_32, 32)
    t2 = _tc_mid(m1, invd, b2.reshape(1, -1), W3)
    m2 = _sc_gather_scatter(t2, srcg, dstg, zeros_16, 16)
    h3, graph_label = _tc_layer3(m2, invd, b3.reshape(1, -1),
                                 Wg1, bg1.reshape(1, -1), Wg2, bg2.reshape(1, -1))
    m3 = _sc_gather_scatter(h3, srcg, dstg, zeros_16, 16)
    node_label = _tc_node(m3, invd, Wn, bn.reshape(1, -1))
    return (graph_label, node_label)


# speedup vs baseline: 1.0640x; 1.0640x over previous
import functools

import jax
import jax.numpy as jnp
from jax import lax
from jax.experimental import pallas as pl
from jax.experimental.pallas import tpu as pltpu
from jax.experimental.pallas import tpu_sc as plsc

_N = 10000
_E = 320000
_NC = 2
_NS = 16
_NW = _NC * _NS
_G = 80
_EPT = _E // _NW
_J = _EPT // _G
_NB = 5
_JO = _J // _NB
_NPAD = 10112
_RPT = _NPAD // _NS


def _sc_mesh():
    return plsc.VectorSubcoreMesh(core_axis_name="c", subcore_axis_name="s")


_NB0 = 5
_G0 = 40
_J0 = _EPT // _G0
_JO0 = _J0 // _NB0


def _sc_pass0(edge_h, dstg, zeros_h, zeros_d, ones_g):
    @functools.partial(
        pl.kernel,
        out_type=(
            jax.ShapeDtypeStruct((_NC, _NPAD, 128), jnp.float32),
            jax.ShapeDtypeStruct((_NC, _NPAD, 16), jnp.float32),
        ),
        mesh=_sc_mesh(),
        scratch_types=(
            pltpu.VMEM((_J0, _G0), jnp.int32),
            pltpu.VMEM((_NB0, _G0, 128), jnp.float32),
            pltpu.VMEM((_G0, 16), jnp.float32),
            pltpu.VMEM_SHARED((_NPAD, 128), jnp.float32),
            pltpu.VMEM_SHARED((_NPAD, 16), jnp.float32),
            pltpu.SemaphoreType.DMA((_NB0,)),
            pltpu.SemaphoreType.DMA((_NB0,)),
            pltpu.SemaphoreType.DMA((_NB0,)),
        ),
        compiler_params=pltpu.CompilerParams(use_tc_tiling_on_sc=False),
    )
    def k(eh, dg, zh, zd, og, oh, od, idxv, ebuf, ones_v, acc_h, acc_d,
          gsem, ssem, dsem):
        cid = lax.axis_index("c")
        sid = lax.axis_index("s")
        wid = cid * _NS + sid
        pltpu.sync_copy(zh, acc_h.at[pl.ds(sid * _RPT, _RPT), :])
        pltpu.sync_copy(zd, acc_d.at[pl.ds(sid * _RPT, _RPT), :])
        pltpu.sync_copy(og, ones_v)
        pltpu.sync_copy(dg.at[wid], idxv)
        plsc.subcore_barrier()

        def edge_dma(j, b):
            base = wid * _EPT + j * _G0
            return pltpu.make_async_copy(
                eh.at[pl.ds(base, _G0), :], ebuf.at[b], gsem.at[b])

        def scat(j, b):
            return (pltpu.make_async_copy(ebuf.at[b], acc_h.at[idxv.at[j]],
                                          ssem.at[b]),
                    pltpu.make_async_copy(ones_v, acc_d.at[idxv.at[j]],
                                          dsem.at[b]))

        for b in range(_NB0):
            edge_dma(b, b).start()

        def body(jo, carry):
            j0 = jo * _NB0
            for b in range(_NB0):
                j = j0 + b
                edge_dma(j, b).wait()
                s1, s2 = scat(j, b)
                s1.start(add=True)
                s2.start(add=True)
            for b in range(_NB0):
                j = j0 + b
                s1, s2 = scat(j, b)
                s1.wait()
                s2.wait()

                @pl.when(jo < _JO0 - 1)
                def _():
                    edge_dma(j + _NB0, b).start()

            return carry

        lax.fori_loop(0, _JO0, body, 0)
        plsc.subcore_barrier()
        pltpu.sync_copy(acc_h.at[pl.ds(sid * _RPT, _RPT), :],
                        oh.at[cid, pl.ds(sid * _RPT, _RPT), :])
        pltpu.sync_copy(acc_d.at[pl.ds(sid * _RPT, _RPT), :],
                        od.at[cid, pl.ds(sid * _RPT, _RPT), :])

    return k(edge_h, dstg, zeros_h, zeros_d, ones_g)


def _sc_gather_scatter(t_nodes, srcg, dstg, zeros_f, f):
    @functools.partial(
        pl.kernel,
        out_type=jax.ShapeDtypeStruct((_NC, _NPAD, f), jnp.float32),
        mesh=_sc_mesh(),
        scratch_types=(
            pltpu.VMEM((_J, _G), jnp.int32),
            pltpu.VMEM((_J, _G), jnp.int32),
            pltpu.VMEM((_NB, _G, f), jnp.float32),
            pltpu.VMEM_SHARED((_NPAD, f), jnp.float32),
            pltpu.SemaphoreType.DMA((_NB,)),
            pltpu.SemaphoreType.DMA((_NB,)),
        ),
        compiler_params=pltpu.CompilerParams(use_tc_tiling_on_sc=False),
    )
    def k(tn, sg, dg, zf, om, sidxv, didxv, rbuf, acc, gsem, ssem):
        cid = lax.axis_index("c")
        sid = lax.axis_index("s")
        wid = cid * _NS + sid
        pltpu.sync_copy(zf, acc.at[pl.ds(sid * _RPT, _RPT), :])
        pltpu.sync_copy(sg.at[wid], sidxv)
        pltpu.sync_copy(dg.at[wid], didxv)
        plsc.subcore_barrier()

        def gather(j, b):
            return pltpu.make_async_copy(
                tn.at[sidxv.at[j]], rbuf.at[b], gsem.at[b])

        for b in range(_NB):
            gather(b, b).start()

        def body(jo, carry):
            j0 = jo * _NB
            for b in range(_NB):
                j = j0 + b
                gather(j, b).wait()
                pltpu.async_copy(rbuf.at[b], acc.at[didxv.at[j]],
                                 ssem.at[b], add=True)
            for b in range(_NB):
                j = j0 + b
                pltpu.make_async_copy(rbuf.at[b], acc.at[didxv.at[j]],
                                      ssem.at[b]).wait()

                @pl.when(jo < _JO - 1)
                def _():
                    gather(j + _NB, b).start()

            return carry

        lax.fori_loop(0, _JO, body, 0)
        plsc.subcore_barrier()
        pltpu.sync_copy(acc.at[pl.ds(sid * _RPT, _RPT), :],
                        om.at[cid, pl.ds(sid * _RPT, _RPT), :])

    return k(t_nodes, srcg, dstg, zeros_f)


def _tc_layer1(acc_h, acc_d, w1, b1, w2):
    def body(ah, ad, w1r, b1r, w2r, t1o, invo):
        s = ah[0, : _N, :] + ah[1, : _N, :]
        d = ad[0, : _N, 0:1] + ad[1, : _N, 0:1]
        inv = 1.0 / jnp.maximum(d, 1.0)
        h1 = jnp.maximum(
            jnp.dot(s * inv, w1r[...], preferred_element_type=jnp.float32)
            + b1r[...], 0.0)
        t1o[...] = jnp.dot(h1, w2r[...], preferred_element_type=jnp.float32)
        invo[...] = inv

    return pl.pallas_call(
        body,
        out_shape=(
            jax.ShapeDtypeStruct((_N, 32), jnp.float32),
            jax.ShapeDtypeStruct((_N, 1), jnp.float32),
        ),
    )(acc_h, acc_d, w1, b1, w2)


def _tc_mid(m, invd, b, wnext):
    f_out = wnext.shape[1]

    def body(mr, ir, br, wr, to):
        s = mr[0, : _N, :] + mr[1, : _N, :]
        h = jnp.maximum(s * ir[...] + br[...], 0.0)
        to[...] = jnp.dot(h, wr[...], preferred_element_type=jnp.float32)

    return pl.pallas_call(
        body,
        out_shape=jax.ShapeDtypeStruct((_N, f_out), jnp.float32),
    )(m, invd, b, wnext)


def _tc_layer3(m, invd, b3, wg1, bg1, wg2, bg2):
    def body(mr, ir, b3r, wg1r, bg1r, wg2r, bg2r, h3o, glo):
        s = mr[0, : _N, :] + mr[1, : _N, :]
        h3 = jnp.maximum(s * ir[...] + b3r[...], 0.0)
        h3o[...] = h3
        gm = jnp.mean(h3, axis=0, keepdims=True)
        z1 = jnp.maximum(
            jnp.dot(gm, wg1r[...], preferred_element_type=jnp.float32)
            + bg1r[...], 0.0)
        z2 = jnp.dot(z1, wg2r[...], preferred_element_type=jnp.float32) + bg2r[...]
        glo[...] = 1.0 / (1.0 + jnp.exp(-z2))

    return pl.pallas_call(
        body,
        out_shape=(
            jax.ShapeDtypeStruct((_N, 16), jnp.float32),
            jax.ShapeDtypeStruct((1, 1), jnp.float32),
        ),
    )(m, invd, b3, wg1, bg1, wg2, bg2)


def _tc_node(m, invd, wn, bn):
    def body(mr, ir, wr, br, no):
        s = mr[0, : _N, :] + mr[1, : _N, :]
        no[...] = (
            jnp.dot(s * ir[...], wr[...], preferred_element_type=jnp.float32)
            + br[...])

    return pl.pallas_call(
        body,
        out_shape=jax.ShapeDtypeStruct((_N, 2), jnp.float32),
    )(m, invd, wn, bn)


@jax.jit
def kernel(edge_h, edge_index, W1, b1, W2, b2, W3, b3, Wn, bn,
           Wg1, bg1, Wg2, bg2):
    srcg = edge_index[0].reshape(_NW, _J, _G)
    dstg = edge_index[1].reshape(_NW, _J, _G)
    dstg0 = edge_index[1].reshape(_NW, _J0, _G0)
    zeros_h = jnp.zeros((_RPT, 128), jnp.float32)
    zeros_d = jnp.zeros((_RPT, 16), jnp.float32)
    zeros_32 = jnp.zeros((_RPT, 32), jnp.float32)
    zeros_16 = jnp.zeros((_RPT, 16), jnp.float32)
    ones_g = jnp.ones((_G0, 16), jnp.float32)

    acc_h, acc_d = _sc_pass0(edge_h, dstg0, zeros_h, zeros_d, ones_g)
    t1, invd = _tc_layer1(acc_h, acc_d, W1, b1.reshape(1, -1), W2)
    m1 = _sc_gather_scatter(t1, srcg, dstg, zeros_32, 32)
    t2 = _tc_mid(m1, invd, b2.reshape(1, -1), W3)
    m2 = _sc_gather_scatter(t2, srcg, dstg, zeros_16, 16)
    h3, graph_label = _tc_layer3(m2, invd, b3.reshape(1, -1),
                                 Wg1, bg1.reshape(1, -1), Wg2, bg2.reshape(1, -1))
    m3 = _sc_gather_scatter(h3, srcg, dstg, zeros_16, 16)
    node_label = _tc_node(m3, invd, Wn, bn.reshape(1, -1))
    return (graph_label, node_label)
